# baseline (speedup 1.0000x reference)
import functools

import jax
import jax.numpy as jnp
from jax import lax
from jax.experimental import pallas as pl
from jax.experimental.pallas import tpu as pltpu
from jax.experimental.pallas import tpu_sc as plsc

VOCAB = 1000000
EMB = 32
BATCH = 16384

L = 16
NC, NS = 2, 16
NW = NC * NS
BPW = BATCH // NW
CHUNK = 128
NCH = BPW // CHUNK
NGROUPS = BPW // L

_mesh = plsc.VectorSubcoreMesh(core_axis_name="c", subcore_axis_name="s")


@functools.partial(
    pl.kernel,
    out_type=jax.ShapeDtypeStruct((BATCH,), jnp.float32),
    mesh=_mesh,
    compiler_params=pltpu.CompilerParams(needs_layout_passes=False,
                                         use_tc_tiling_on_sc=False),
    scratch_types=[
        pltpu.VMEM((BPW,), jnp.int32),
        pltpu.VMEM((BPW,), jnp.int32),
        pltpu.VMEM((EMB, BPW), jnp.float32),
        pltpu.VMEM((EMB, BPW), jnp.float32),
        pltpu.VMEM((BPW,), jnp.float32),
        pltpu.VMEM((BPW,), jnp.float32),
        pltpu.VMEM((L,), jnp.float32),
        pltpu.VMEM((BPW,), jnp.float32),
        pltpu.SemaphoreType.DMA,
    ],
)
def _mf_kernel(row_id_hbm, col_id_hbm, rembt_hbm, cembt_hbm, rbias_hbm,
               cbias_hbm, gb_hbm, out_hbm,
               ridx_v, cidx_v, rrows_v, crows_v, rb_v, cb_v, gb_v, out_v,
               sem):
    wid = lax.axis_index("s") * NC + lax.axis_index("c")

    pltpu.sync_copy(row_id_hbm.at[wid], ridx_v)
    pltpu.sync_copy(col_id_hbm.at[wid], cidx_v)
    pltpu.sync_copy(gb_hbm, gb_v)

    def gather_g(g, carry):
        sl = pl.ds(g * L, L)
        rid16 = ridx_v[sl]
        cid16 = cidx_v[sl]
        pltpu.async_copy(rbias_hbm.at[rid16], rb_v.at[sl], sem)
        pltpu.async_copy(cbias_hbm.at[cid16], cb_v.at[sl], sem)
        for e in range(EMB):
            pltpu.async_copy(rembt_hbm.at[e].at[rid16], rrows_v.at[e, sl], sem)
            pltpu.async_copy(cembt_hbm.at[e].at[cid16], crows_v.at[e, sl], sem)
        return carry

    lax.fori_loop(0, NGROUPS, gather_g, 0)

    def wait_g(g, carry):
        sl = pl.ds(g * L, L)
        rid16 = ridx_v[sl]
        cid16 = cidx_v[sl]
        pltpu.make_async_copy(rbias_hbm.at[rid16], rb_v.at[sl], sem).wait()
        pltpu.make_async_copy(cbias_hbm.at[cid16], cb_v.at[sl], sem).wait()
        for e in range(EMB):
            pltpu.make_async_copy(rembt_hbm.at[e].at[rid16],
                                  rrows_v.at[e, sl], sem).wait()
            pltpu.make_async_copy(cembt_hbm.at[e].at[cid16],
                                  crows_v.at[e, sl], sem).wait()
        return carry

    lax.fori_loop(0, NGROUPS, wait_g, 0)

    gbv = gb_v[...]

    def group(g, carry):
        sl = pl.ds(g * L, L)
        acc = rb_v[sl] + cb_v[sl] + gbv
        for e in range(EMB):
            acc = acc + rrows_v[e, sl] * crows_v[e, sl]
        out_v[sl] = acc
        return carry

    lax.fori_loop(0, NGROUPS, group, 0)
    pltpu.sync_copy(out_v, out_hbm.at[pl.ds(wid * BPW, BPW)])


def kernel(row_id, col_id, row_emb_table, col_emb_table, row_bias_table,
           col_bias_table, global_bias):
    rid = row_id.astype(jnp.int32).reshape(NW, BPW)
    cid = col_id.astype(jnp.int32).reshape(NW, BPW)
    rembt = row_emb_table.T
    cembt = col_emb_table.T
    rb = row_bias_table.reshape(VOCAB)
    cb = col_bias_table.reshape(VOCAB)
    gb = jnp.broadcast_to(global_bias.astype(jnp.float32), (L,))
    out = _mf_kernel(rid, cid, rembt, cembt, rb, cb, gb)
    return out.reshape(BATCH, 1)


# speedup vs baseline: 5.5336x; 5.5336x over previous
import functools

import jax
import jax.numpy as jnp
from jax import lax
from jax.experimental import pallas as pl
from jax.experimental.pallas import tpu as pltpu
from jax.experimental.pallas import tpu_sc as plsc

VOCAB = 1000000
EMB = 32
BATCH = 16384

L = 16
NC, NS = 2, 16
NW = NC * NS
BPW = BATCH // NW
CHUNK = 128
NCH = BPW // CHUNK
NGROUPS = BPW // L

_mesh = plsc.VectorSubcoreMesh(core_axis_name="c", subcore_axis_name="s")


@functools.partial(
    pl.kernel,
    out_type=jax.ShapeDtypeStruct((BATCH,), jnp.float32),
    mesh=_mesh,
    compiler_params=pltpu.CompilerParams(needs_layout_passes=False,
                                         use_tc_tiling_on_sc=False),
    scratch_types=[
        pltpu.VMEM((BPW,), jnp.int32),
        pltpu.VMEM((BPW,), jnp.int32),
        pltpu.VMEM((EMB, BPW), jnp.float32),
        pltpu.VMEM((EMB, BPW), jnp.float32),
        pltpu.VMEM((BPW,), jnp.float32),
        pltpu.VMEM((BPW,), jnp.float32),
        pltpu.VMEM((L,), jnp.float32),
        pltpu.VMEM((BPW,), jnp.float32),
        pltpu.SemaphoreType.DMA,
    ],
)
def _mf_kernel(row_id_hbm, col_id_hbm, rembt_hbm, cembt_hbm, rbias_hbm,
               cbias_hbm, gb_hbm, out_hbm,
               ridx_v, cidx_v, rrows_v, crows_v, rb_v, cb_v, gb_v, out_v,
               sem):
    wid = lax.axis_index("s") * NC + lax.axis_index("c")

    pltpu.sync_copy(row_id_hbm.at[wid], ridx_v)
    pltpu.sync_copy(col_id_hbm.at[wid], cidx_v)
    pltpu.sync_copy(gb_hbm, gb_v)

    def gather_g(g, carry):
        sl = pl.ds(g * L, L)
        rid16 = ridx_v[sl]
        cid16 = cidx_v[sl]
        pltpu.async_copy(rbias_hbm.at[rid16], rb_v.at[sl], sem)
        pltpu.async_copy(cbias_hbm.at[cid16], cb_v.at[sl], sem)
        rflat = rid16 * EMB
        cflat = cid16 * EMB
        for e in range(EMB):
            pltpu.async_copy(rembt_hbm.at[rflat + e], rrows_v.at[e, sl], sem)
            pltpu.async_copy(cembt_hbm.at[cflat + e], crows_v.at[e, sl], sem)
        return carry

    lax.fori_loop(0, NGROUPS, gather_g, 0)

    def wait_g(g, carry):
        sl = pl.ds(g * L, L)
        rid16 = ridx_v[sl]
        cid16 = cidx_v[sl]
        pltpu.make_async_copy(rbias_hbm.at[rid16], rb_v.at[sl], sem).wait()
        pltpu.make_async_copy(cbias_hbm.at[cid16], cb_v.at[sl], sem).wait()
        rflat = rid16 * EMB
        cflat = cid16 * EMB
        for e in range(EMB):
            pltpu.make_async_copy(rembt_hbm.at[rflat + e],
                                  rrows_v.at[e, sl], sem).wait()
            pltpu.make_async_copy(cembt_hbm.at[cflat + e],
                                  crows_v.at[e, sl], sem).wait()
        return carry

    lax.fori_loop(0, NGROUPS, wait_g, 0)

    gbv = gb_v[...]

    def group(g, carry):
        sl = pl.ds(g * L, L)
        acc = rb_v[sl] + cb_v[sl] + gbv
        for e in range(EMB):
            acc = acc + rrows_v[e, sl] * crows_v[e, sl]
        out_v[sl] = acc
        return carry

    lax.fori_loop(0, NGROUPS, group, 0)
    pltpu.sync_copy(out_v, out_hbm.at[pl.ds(wid * BPW, BPW)])


def kernel(row_id, col_id, row_emb_table, col_emb_table, row_bias_table,
           col_bias_table, global_bias):
    rid = row_id.astype(jnp.int32).reshape(NW, BPW)
    cid = col_id.astype(jnp.int32).reshape(NW, BPW)
    rembt = row_emb_table.reshape(VOCAB * EMB)
    cembt = col_emb_table.reshape(VOCAB * EMB)
    rb = row_bias_table.reshape(VOCAB)
    cb = col_bias_table.reshape(VOCAB)
    gb = jnp.broadcast_to(global_bias.astype(jnp.float32), (L,))
    out = _mf_kernel(rid, cid, rembt, cembt, rb, cb, gb)
    return out.reshape(BATCH, 1)


# speedup vs baseline: 5.6945x; 1.0291x over previous
import functools

import jax
import jax.numpy as jnp
from jax import lax
from jax.experimental import pallas as pl
from jax.experimental.pallas import tpu as pltpu
from jax.experimental.pallas import tpu_sc as plsc

VOCAB = 1000000
EMB = 32
BATCH = 16384

L = 16
NC, NS = 2, 16
NW = NC * NS
BPW = BATCH // NW
CHUNK = 128
NCH = BPW // CHUNK
GPC = CHUNK // L
ROWS = VOCAB * EMB // 128

_mesh = plsc.VectorSubcoreMesh(core_axis_name="c", subcore_axis_name="s")


@functools.partial(
    pl.kernel,
    out_type=jax.ShapeDtypeStruct((BATCH,), jnp.float32),
    mesh=_mesh,
    compiler_params=pltpu.CompilerParams(needs_layout_passes=False,
                                         use_tc_tiling_on_sc=False),
    scratch_types=[
        pltpu.VMEM((BPW,), jnp.int32),
        pltpu.VMEM((BPW,), jnp.int32),
        pltpu.VMEM((NCH, CHUNK), jnp.int32),
        pltpu.VMEM((NCH, CHUNK), jnp.int32),
        pltpu.VMEM((2, CHUNK, 128), jnp.float32),
        pltpu.VMEM((2, CHUNK, 128), jnp.float32),
        pltpu.VMEM((BPW,), jnp.float32),
        pltpu.VMEM((BPW,), jnp.float32),
        pltpu.VMEM((L,), jnp.float32),
        pltpu.VMEM((BPW,), jnp.float32),
        pltpu.SemaphoreType.DMA,
        pltpu.SemaphoreType.DMA,
        pltpu.SemaphoreType.DMA,
    ],
)
def _mf_kernel(row_id_hbm, col_id_hbm, rrow_hbm, crow_hbm, remb_hbm,
               cemb_hbm, rbias_hbm, cbias_hbm, gb_hbm, out_hbm,
               ridx_v, cidx_v, rrow_v, crow_v, rbuf_v, cbuf_v, rb_v, cb_v,
               gb_v, out_v, esem0, esem1, bsem):
    esems = (esem0, esem1)
    wid = lax.axis_index("s") * NC + lax.axis_index("c")

    pltpu.sync_copy(row_id_hbm.at[wid], ridx_v)
    pltpu.sync_copy(col_id_hbm.at[wid], cidx_v)
    pltpu.sync_copy(rrow_hbm.at[wid], rrow_v)
    pltpu.sync_copy(crow_hbm.at[wid], crow_v)
    pltpu.sync_copy(gb_hbm, gb_v)

    iota16 = lax.iota(jnp.int32, L)

    def bias_g(g, carry):
        sl = pl.ds(g * L, L)
        pltpu.async_copy(rbias_hbm.at[ridx_v[sl]], rb_v.at[sl], bsem)
        pltpu.async_copy(cbias_hbm.at[cidx_v[sl]], cb_v.at[sl], bsem)
        return carry

    lax.fori_loop(0, BPW // L, bias_g, 0)

    def issue(j, buf):
        pltpu.async_copy(remb_hbm.at[rrow_v.at[j]], rbuf_v.at[buf],
                         esems[buf])
        pltpu.async_copy(cemb_hbm.at[crow_v.at[j]], cbuf_v.at[buf],
                         esems[buf])

    def wait(j, buf):
        pltpu.make_async_copy(remb_hbm.at[rrow_v.at[j]], rbuf_v.at[buf],
                              esems[buf]).wait()
        pltpu.make_async_copy(cemb_hbm.at[crow_v.at[j]], cbuf_v.at[buf],
                              esems[buf]).wait()

    def compute(j, buf):
        for g in range(GPC):
            b0 = j * CHUNK + g * L
            sl = pl.ds(b0, L)
            lane = g * L + iota16
            rw = lax.shift_left(ridx_v[sl] & 3, 5)
            cw = lax.shift_left(cidx_v[sl] & 3, 5)
            acc = rb_v[sl] + cb_v[sl] + gb_v[...]
            for e in range(EMB):
                r = plsc.load_gather(rbuf_v, [jnp.full((L,), buf, jnp.int32),
                                              lane, rw + e])
                c = plsc.load_gather(cbuf_v, [jnp.full((L,), buf, jnp.int32),
                                              lane, cw + e])
                acc = acc + r * c
            out_v[sl] = acc

    issue(0, 0)
    issue(1, 1)

    def bias_w(g, carry):
        sl = pl.ds(g * L, L)
        pltpu.make_async_copy(rbias_hbm.at[ridx_v[sl]], rb_v.at[sl],
                              bsem).wait()
        pltpu.make_async_copy(cbias_hbm.at[cidx_v[sl]], cb_v.at[sl],
                              bsem).wait()
        return carry

    lax.fori_loop(0, BPW // L, bias_w, 0)

    wait(0, 0)
    compute(0, 0)
    issue(2, 0)
    wait(1, 1)
    compute(1, 1)
    issue(3, 1)
    wait(2, 0)
    compute(2, 0)
    wait(3, 1)
    compute(3, 1)

    pltpu.sync_copy(out_v, out_hbm.at[pl.ds(wid * BPW, BPW)])


def kernel(row_id, col_id, row_emb_table, col_emb_table, row_bias_table,
           col_bias_table, global_bias):
    rid32 = row_id.astype(jnp.int32)
    cid32 = col_id.astype(jnp.int32)
    rid = rid32.reshape(NW, BPW)
    cid = cid32.reshape(NW, BPW)
    rrow = (rid32 // 4).reshape(NW, NCH, CHUNK)
    crow = (cid32 // 4).reshape(NW, NCH, CHUNK)
    rembt = row_emb_table.reshape(ROWS, 128)
    cembt = col_emb_table.reshape(ROWS, 128)
    rb = row_bias_table.reshape(VOCAB)
    cb = col_bias_table.reshape(VOCAB)
    gb = jnp.broadcast_to(global_bias.astype(jnp.float32), (L,))
    out = _mf_kernel(rid, cid, rrow, crow, rembt, cembt, rb, cb, gb)
    return out.reshape(BATCH, 1)


# speedup vs baseline: 7.6507x; 1.3435x over previous
import functools

import jax
import jax.numpy as jnp
from jax import lax
from jax.experimental import pallas as pl
from jax.experimental.pallas import tpu as pltpu
from jax.experimental.pallas import tpu_sc as plsc

VOCAB = 1000000
EMB = 32
BATCH = 16384

L = 16
NC, NS = 2, 16
NW = NC * NS
BPW = BATCH // NW
CHUNK = 128
NCH = BPW // CHUNK
GPC = CHUNK // L
ROWS = VOCAB * EMB // 128

_mesh = plsc.VectorSubcoreMesh(core_axis_name="c", subcore_axis_name="s")


HALF = CHUNK


@functools.partial(
    pl.kernel,
    out_type=jax.ShapeDtypeStruct((BATCH,), jnp.float32),
    mesh=_mesh,
    compiler_params=pltpu.CompilerParams(needs_layout_passes=False),
    scratch_types=[
        pltpu.VMEM((BPW,), jnp.int32),
        pltpu.VMEM((BPW,), jnp.int32),
        pltpu.VMEM((2, HALF, EMB), jnp.float32),
        pltpu.VMEM((2, HALF, EMB), jnp.float32),
        pltpu.VMEM((L,), jnp.float32),
        pltpu.VMEM((BPW,), jnp.float32),
        pltpu.SemaphoreType.DMA,
        pltpu.SemaphoreType.DMA,
    ],
)
def _dot_kernel(row_id_hbm, col_id_hbm, remb_hbm, cemb_hbm, gb_hbm, out_hbm,
                ridx_v, cidx_v, rbuf_v, cbuf_v, gb_v, out_v, esem0, esem1):
    esems = (esem0, esem1)
    wid = lax.axis_index("s") * NC + lax.axis_index("c")

    pltpu.sync_copy(row_id_hbm.at[wid], ridx_v)
    pltpu.sync_copy(col_id_hbm.at[wid], cidx_v)
    pltpu.sync_copy(gb_hbm, gb_v)

    iota16 = lax.iota(jnp.int32, L)

    def issue(h, buf):
        def grp(g, carry):
            rid16 = ridx_v[pl.ds(h * HALF + g * L, L)]
            cid16 = cidx_v[pl.ds(h * HALF + g * L, L)]
            for i in range(L):
                slot = pl.ds(g * L + i, 1)
                pltpu.async_copy(remb_hbm.at[pl.ds(rid16[i], 1), :],
                                 rbuf_v.at[buf, slot, :], esems[buf])
                pltpu.async_copy(cemb_hbm.at[pl.ds(cid16[i], 1), :],
                                 cbuf_v.at[buf, slot, :], esems[buf])
            return carry

        lax.fori_loop(0, HALF // L, grp, 0)

    def wait(buf):
        pltpu.make_async_copy(remb_hbm.at[pl.ds(0, HALF), :],
                              rbuf_v.at[buf], esems[buf]).wait()
        pltpu.make_async_copy(cemb_hbm.at[pl.ds(0, HALF), :],
                              cbuf_v.at[buf], esems[buf]).wait()

    def compute(h, buf):
        bufv = jnp.full((L,), buf, jnp.int32)

        def grp(g, carry):
            slot16 = g * L + iota16
            acc = gb_v[...]
            for e in range(EMB):
                ev = jnp.full((L,), e, jnp.int32)
                r = plsc.load_gather(rbuf_v, [bufv, slot16, ev])
                c = plsc.load_gather(cbuf_v, [bufv, slot16, ev])
                acc = acc + r * c
            out_v[pl.ds(h * HALF + g * L, L)] = acc
            return carry

        lax.fori_loop(0, HALF // L, grp, 0)

    issue(0, 0)
    issue(1, 1)
    wait(0)
    compute(0, 0)
    issue(2, 0)
    wait(1)
    compute(1, 1)
    issue(3, 1)
    wait(0)
    compute(2, 0)
    wait(1)
    compute(3, 1)

    pltpu.sync_copy(out_v, out_hbm.at[pl.ds(wid * BPW, BPW)])


@functools.partial(
    pl.kernel,
    out_type=jax.ShapeDtypeStruct((BATCH,), jnp.float32),
    mesh=_mesh,
    compiler_params=pltpu.CompilerParams(needs_layout_passes=False,
                                         use_tc_tiling_on_sc=False),
    scratch_types=[
        pltpu.VMEM((BPW,), jnp.int32),
        pltpu.VMEM((BPW,), jnp.int32),
        pltpu.VMEM((BPW,), jnp.float32),
        pltpu.VMEM((BPW,), jnp.float32),
        pltpu.SemaphoreType.DMA,
    ],
)
def _bias_kernel(row_id_hbm, col_id_hbm, rbias_hbm, cbias_hbm, out_hbm,
                 ridx_v, cidx_v, rb_v, cb_v, bsem):
    wid = lax.axis_index("s") * NC + lax.axis_index("c")

    pltpu.sync_copy(row_id_hbm.at[wid], ridx_v)
    pltpu.sync_copy(col_id_hbm.at[wid], cidx_v)

    def bias_g(g, carry):
        sl = pl.ds(g * L, L)
        pltpu.async_copy(rbias_hbm.at[ridx_v[sl]], rb_v.at[sl], bsem)
        pltpu.async_copy(cbias_hbm.at[cidx_v[sl]], cb_v.at[sl], bsem)
        return carry

    lax.fori_loop(0, BPW // L, bias_g, 0)

    def bias_w(g, carry):
        sl = pl.ds(g * L, L)
        pltpu.make_async_copy(rbias_hbm.at[ridx_v[sl]], rb_v.at[sl],
                              bsem).wait()
        pltpu.make_async_copy(cbias_hbm.at[cidx_v[sl]], cb_v.at[sl],
                              bsem).wait()
        return carry

    lax.fori_loop(0, BPW // L, bias_w, 0)

    def bias_sum(g, carry):
        sl = pl.ds(g * L, L)
        rb_v[sl] = rb_v[sl] + cb_v[sl]
        return carry

    lax.fori_loop(0, BPW // L, bias_sum, 0)
    pltpu.sync_copy(rb_v, out_hbm.at[pl.ds(wid * BPW, BPW)])


def kernel(row_id, col_id, row_emb_table, col_emb_table, row_bias_table,
           col_bias_table, global_bias):
    rid = row_id.astype(jnp.int32).reshape(NW, BPW)
    cid = col_id.astype(jnp.int32).reshape(NW, BPW)
    rb = row_bias_table.reshape(VOCAB)
    cb = col_bias_table.reshape(VOCAB)
    gb = jnp.broadcast_to(global_bias.astype(jnp.float32), (L,))
    dot = _dot_kernel(rid, cid, row_emb_table, col_emb_table, gb)
    bias = _bias_kernel(rid, cid, rb, cb)
    return (dot + bias).reshape(BATCH, 1)
